# baseline (device time: 41790 ns/iter reference)
import jax
import jax.numpy as jnp
from jax import lax
from jax.experimental import pallas as pl
from jax.experimental.pallas import tpu as pltpu

R = 256
MAXC = 2048 // R


def kernel(x, dest):
    m, n = x.shape
    my_y = lax.axis_index("y")

    iota = jnp.arange(m, dtype=jnp.int32)
    is0 = dest == 0
    cum = jnp.cumsum(is0.astype(jnp.int32))
    c0 = cum[m - 1]
    pos_in_group = jnp.where(is0, cum - 1, iota - cum)
    is_send = jnp.where(my_y == 0, ~is0, is0)
    c_keep = jnp.where(my_y == 0, c0, m - c0)
    rc = m - c_keep
    n_c = (rc + R - 1) // R
    o = jnp.where(my_y == 0, 0, n_c * R - rc)
    keep_slot = jnp.where(my_y == 0, pos_in_group, rc + pos_in_group)
    slot = jnp.where(is_send, m + o + pos_in_group, keep_slot)

    def body(c0_ref, slot_ref, x_ref, out_ref, xs_ref, send_sems, recv_sems):
        my_x = lax.axis_index("x")
        yy = lax.axis_index("y")
        peer = 1 - yy
        c0_ = c0_ref[0]
        c_keep_ = jnp.where(yy == 0, c0_, m - c0_)
        rc_ = m - c_keep_
        n_c_ = (rc_ + R - 1) // R

        barrier_sem = pltpu.get_barrier_semaphore()
        pl.semaphore_signal(
            barrier_sem,
            inc=1,
            device_id=(my_x, peer),
            device_id_type=pl.DeviceIdType.MESH,
        )
        pl.semaphore_wait(barrier_sem, 1)

        xf = x_ref[...]
        slot_row = slot_ref[...]

        def permute_block(base):
            pk = lax.broadcasted_iota(jnp.int32, (R, m), 0) + base
            onehot = (slot_row == pk).astype(jnp.float32)
            blk = jnp.dot(onehot, xf, preferred_element_type=jnp.float32)
            xs_ref[pl.ds(base, R), :] = blk.astype(jnp.bfloat16)

        for k in range(MAXC):
            @pl.when(k < n_c_)
            def _(k=k):
                permute_block(m + k * R)
                dst0 = jnp.where(yy == 0, k * R, m - n_c_ * R + k * R)
                pltpu.make_async_remote_copy(
                    src_ref=xs_ref.at[pl.ds(m + k * R, R)],
                    dst_ref=out_ref.at[pl.ds(pl.multiple_of(dst0, R), R)],
                    send_sem=send_sems.at[k],
                    recv_sem=recv_sems.at[k],
                    device_id=(my_x, peer),
                    device_id_type=pl.DeviceIdType.MESH,
                ).start()

        for kb in range(MAXC):
            guard = jnp.where(
                yy == 0, kb * R < c_keep_, (kb + 1) * R > rc_
            )

            @pl.when(guard)
            def _(kb=kb):
                permute_block(kb * R)

        for k in range(MAXC):
            @pl.when(k < n_c_)
            def _(k=k):
                dst0 = jnp.where(yy == 0, k * R, m - n_c_ * R + k * R)
                rdma = pltpu.make_async_remote_copy(
                    src_ref=xs_ref.at[pl.ds(m + k * R, R)],
                    dst_ref=out_ref.at[pl.ds(pl.multiple_of(dst0, R), R)],
                    send_sem=send_sems.at[k],
                    recv_sem=recv_sems.at[k],
                    device_id=(my_x, peer),
                    device_id_type=pl.DeviceIdType.MESH,
                )
                rdma.wait_send()
                rdma.wait_recv()

        row = lax.broadcasted_iota(jnp.int32, (m, 1), 0)

        @pl.when(yy == 0)
        def _():
            out_ref[...] = jnp.where(
                row < c_keep_, xs_ref[pl.ds(0, m), :], out_ref[...]
            )

        @pl.when(yy == 1)
        def _():
            out_ref[...] = jnp.where(
                row >= rc_, xs_ref[pl.ds(0, m), :], out_ref[...]
            )

    return pl.pallas_call(
        body,
        out_shape=jax.ShapeDtypeStruct((m, n), jnp.bfloat16),
        in_specs=[
            pl.BlockSpec(memory_space=pltpu.SMEM),
            pl.BlockSpec(memory_space=pltpu.VMEM),
            pl.BlockSpec(memory_space=pltpu.VMEM),
        ],
        out_specs=pl.BlockSpec(memory_space=pltpu.VMEM),
        scratch_shapes=[
            pltpu.VMEM((2 * m, n), jnp.bfloat16),
            pltpu.SemaphoreType.DMA((MAXC,)),
            pltpu.SemaphoreType.DMA((MAXC,)),
        ],
        compiler_params=pltpu.CompilerParams(
            collective_id=0, vmem_limit_bytes=64 * 1024 * 1024
        ),
    )(jnp.reshape(c0, (1,)), jnp.reshape(slot, (1, m)), x)


# device time: 41403 ns/iter; 1.0093x vs baseline; 1.0093x over previous
import jax
import jax.numpy as jnp
from jax import lax
from jax.experimental import pallas as pl
from jax.experimental.pallas import tpu as pltpu

R = 256
MAXC = 2048 // R


def kernel(x, dest):
    m, n = x.shape
    my_y = lax.axis_index("y")

    iota = jnp.arange(m, dtype=jnp.int32)
    is0 = dest == 0
    cum = jnp.cumsum(is0.astype(jnp.int32))
    c0 = cum[m - 1]
    pos_in_group = jnp.where(is0, cum - 1, iota - cum)
    is_send = jnp.where(my_y == 0, ~is0, is0)
    c_keep = jnp.where(my_y == 0, c0, m - c0)
    rc = m - c_keep
    n_c = (rc + R - 1) // R
    o = jnp.where(my_y == 0, 0, n_c * R - rc)
    keep_slot = jnp.where(my_y == 0, pos_in_group, rc + pos_in_group)
    slot = jnp.where(is_send, m + o + pos_in_group, keep_slot)

    def body(c0_ref, slot_ref, x_ref, out_ref, xs_ref, send_sems, recv_sems):
        my_x = lax.axis_index("x")
        yy = lax.axis_index("y")
        peer = 1 - yy
        c0_ = c0_ref[0]
        c_keep_ = jnp.where(yy == 0, c0_, m - c0_)
        rc_ = m - c_keep_
        n_c_ = (rc_ + R - 1) // R

        barrier_sem = pltpu.get_barrier_semaphore()
        pl.semaphore_signal(
            barrier_sem,
            inc=1,
            device_id=(my_x, peer),
            device_id_type=pl.DeviceIdType.MESH,
        )
        pl.semaphore_wait(barrier_sem, 1)

        xf = x_ref[...]
        slot_row = slot_ref[...]

        def permute_block(base, dst_ref, dst_base):
            pk = lax.broadcasted_iota(jnp.int32, (R, m), 0) + base
            onehot = (slot_row == pk).astype(jnp.float32)
            blk = jnp.dot(onehot, xf, preferred_element_type=jnp.float32)
            dst_ref[pl.ds(dst_base, R), :] = blk.astype(jnp.bfloat16)

        for k in range(MAXC):
            @pl.when(k < n_c_)
            def _(k=k):
                permute_block(m + k * R, xs_ref, m + k * R)
                dst0 = jnp.where(yy == 0, k * R, m - n_c_ * R + k * R)
                pltpu.make_async_remote_copy(
                    src_ref=xs_ref.at[pl.ds(m + k * R, R)],
                    dst_ref=out_ref.at[pl.ds(pl.multiple_of(dst0, R), R)],
                    send_sem=send_sems.at[k],
                    recv_sem=recv_sems.at[k],
                    device_id=(my_x, peer),
                    device_id_type=pl.DeviceIdType.MESH,
                ).start()

        for kb in range(MAXC):
            direct = jnp.where(
                yy == 0, kb + 1 + n_c_ <= MAXC, kb >= n_c_
            )
            fringe = jnp.where(
                yy == 0,
                jnp.logical_and(kb + 1 + n_c_ > MAXC, kb * R < c_keep_),
                jnp.logical_and(kb < n_c_, (kb + 1) * R > rc_),
            )

            @pl.when(direct)
            def _(kb=kb):
                permute_block(kb * R, out_ref, kb * R)

            @pl.when(fringe)
            def _(kb=kb):
                permute_block(kb * R, xs_ref, kb * R)

        for k in range(MAXC):
            @pl.when(k < n_c_)
            def _(k=k):
                dst0 = jnp.where(yy == 0, k * R, m - n_c_ * R + k * R)
                rdma = pltpu.make_async_remote_copy(
                    src_ref=xs_ref.at[pl.ds(m + k * R, R)],
                    dst_ref=out_ref.at[pl.ds(pl.multiple_of(dst0, R), R)],
                    send_sem=send_sems.at[k],
                    recv_sem=recv_sems.at[k],
                    device_id=(my_x, peer),
                    device_id_type=pl.DeviceIdType.MESH,
                )
                rdma.wait_send()
                rdma.wait_recv()

        for kb in range(MAXC):
            fringe = jnp.where(
                yy == 0,
                jnp.logical_and(kb + 1 + n_c_ > MAXC, kb * R < c_keep_),
                jnp.logical_and(kb < n_c_, (kb + 1) * R > rc_),
            )

            @pl.when(jnp.logical_and(fringe, yy == 0))
            def _(kb=kb):
                rowb = lax.broadcasted_iota(jnp.int32, (R, 1), 0) + kb * R
                out_ref[pl.ds(kb * R, R), :] = jnp.where(
                    rowb < c_keep_,
                    xs_ref[pl.ds(kb * R, R), :],
                    out_ref[pl.ds(kb * R, R), :],
                )

            @pl.when(jnp.logical_and(fringe, yy == 1))
            def _(kb=kb):
                rowb = lax.broadcasted_iota(jnp.int32, (R, 1), 0) + kb * R
                out_ref[pl.ds(kb * R, R), :] = jnp.where(
                    rowb >= rc_,
                    xs_ref[pl.ds(kb * R, R), :],
                    out_ref[pl.ds(kb * R, R), :],
                )

    return pl.pallas_call(
        body,
        out_shape=jax.ShapeDtypeStruct((m, n), jnp.bfloat16),
        in_specs=[
            pl.BlockSpec(memory_space=pltpu.SMEM),
            pl.BlockSpec(memory_space=pltpu.VMEM),
            pl.BlockSpec(memory_space=pltpu.VMEM),
        ],
        out_specs=pl.BlockSpec(memory_space=pltpu.VMEM),
        scratch_shapes=[
            pltpu.VMEM((2 * m, n), jnp.bfloat16),
            pltpu.SemaphoreType.DMA((MAXC,)),
            pltpu.SemaphoreType.DMA((MAXC,)),
        ],
        compiler_params=pltpu.CompilerParams(
            collective_id=0, vmem_limit_bytes=64 * 1024 * 1024
        ),
    )(jnp.reshape(c0, (1,)), jnp.reshape(slot, (1, m)), x)
